# baseline (speedup 1.0000x reference)
import jax
import jax.numpy as jnp
from jax import lax
from jax.experimental import pallas as pl
from jax.experimental.pallas import tpu as pltpu
from jax.experimental.pallas import tpu_sc as plsc

NC, NS = 2, 16
NW = NC * NS
CHUNK = 128


def kernel(x, table):
    B, H = x.shape
    V, D = table.shape
    N = B * H
    n_chunks = N // CHUNK
    per_w = n_chunks // NW
    assert n_chunks * CHUNK == N and per_w * NW == n_chunks

    xf = x.reshape(n_chunks, CHUNK).astype(jnp.int32)
    mesh = plsc.VectorSubcoreMesh(
        core_axis_name="c", subcore_axis_name="s",
        num_cores=NC, num_subcores=NS,
    )

    def body(idx_hbm, table_hbm, out_hbm, idx_v, rows_v, gsem):
        wid = lax.axis_index("s") * NC + lax.axis_index("c")
        cbase = wid * per_w
        pltpu.sync_copy(idx_hbm.at[pl.ds(cbase, per_w)], idx_v)

        def step(j, carry):
            pltpu.async_copy(table_hbm.at[idx_v.at[j]], rows_v, gsem).wait()
            pltpu.sync_copy(
                rows_v, out_hbm.at[pl.ds((cbase + j) * CHUNK, CHUNK)])
            return carry

        lax.fori_loop(0, per_w, step, 0)

    out = pl.kernel(
        body,
        out_type=jax.ShapeDtypeStruct((N, D), jnp.float32),
        mesh=mesh,
        compiler_params=pltpu.CompilerParams(use_tc_tiling_on_sc=False),
        scratch_types=[
            pltpu.VMEM((per_w, CHUNK), jnp.int32),
            pltpu.VMEM((CHUNK, D), jnp.float32),
            pltpu.SemaphoreType.DMA,
        ],
    )(xf, table)
    return out.reshape(B, H, D)


# speedup vs baseline: 1.1118x; 1.1118x over previous
import jax
import jax.numpy as jnp
from jax import lax
from jax.experimental import pallas as pl
from jax.experimental.pallas import tpu as pltpu
from jax.experimental.pallas import tpu_sc as plsc

NC, NS = 2, 16
NW = NC * NS
CHUNK = 128


def kernel(x, table):
    B, H = x.shape
    V, D = table.shape
    N = B * H
    n_chunks = N // CHUNK
    per_w = n_chunks // NW
    assert n_chunks * CHUNK == N and per_w * NW == n_chunks

    xf = x.reshape(n_chunks, CHUNK).astype(jnp.int32)
    mesh = plsc.VectorSubcoreMesh(
        core_axis_name="c", subcore_axis_name="s",
        num_cores=NC, num_subcores=NS,
    )

    NBUF = 8
    KMAX = per_w // NBUF
    assert KMAX * NBUF == per_w

    def body(idx_hbm, table_hbm, out_hbm, idx_v, rows_v, *sems):
        gsem, wsem = sems[:NBUF], sems[NBUF:]
        wid = lax.axis_index("s") * NC + lax.axis_index("c")
        cbase = wid * per_w
        pltpu.sync_copy(idx_hbm.at[pl.ds(cbase, per_w)], idx_v)

        for b in range(NBUF):
            pltpu.async_copy(
                table_hbm.at[idx_v.at[b]], rows_v.at[b], gsem[b])

        def step(k, carry):
            for b in range(NBUF):
                j = k * NBUF + b
                pltpu.make_async_copy(
                    table_hbm.at[idx_v.at[0]], rows_v.at[b], gsem[b]).wait()
                pltpu.async_copy(
                    rows_v.at[b],
                    out_hbm.at[pl.ds((cbase + j) * CHUNK, CHUNK)], wsem[b])
            for b in range(NBUF):
                pltpu.make_async_copy(
                    rows_v.at[b], out_hbm.at[pl.ds(0, CHUNK)], wsem[b]).wait()

                @pl.when(k < KMAX - 1)
                def _():
                    pltpu.async_copy(
                        table_hbm.at[idx_v.at[(k + 1) * NBUF + b]],
                        rows_v.at[b], gsem[b])
            return carry

        lax.fori_loop(0, KMAX, step, 0)

    out = pl.kernel(
        body,
        out_type=jax.ShapeDtypeStruct((N, D), jnp.float32),
        mesh=mesh,
        compiler_params=pltpu.CompilerParams(use_tc_tiling_on_sc=False),
        scratch_types=[
            pltpu.VMEM((per_w, CHUNK), jnp.int32),
            pltpu.VMEM((NBUF, CHUNK, D), jnp.float32),
        ] + [pltpu.SemaphoreType.DMA] * (2 * NBUF),
    )(xf, table)
    return out.reshape(B, H, D)
